# baseline (speedup 1.0000x reference)
import functools
import math

import jax
import jax.numpy as jnp
from jax import lax
from jax.experimental import pallas as pl
from jax.experimental.pallas import tpu as pltpu
from jax.experimental.pallas import tpu_sc as plsc

NUM_BUCKETS = 32
MAX_DISTANCE = 128
N_HEADS = 16

Q_LEN = 2048
K_LEN = 2048
N_RHO = 16
N_K0 = 16
SLAB_W = 128 * (N_K0 - 1) + K_LEN
C_SRC = 4224


def _table_body(wt_ref, out_ref, vtab_ref):
    c = lax.broadcasted_iota(jnp.int32, (1, C_SRC), 1)
    d = c - (Q_LEN - 1)
    nb = NUM_BUCKETS // 2
    max_exact = nb // 2
    bucket = jnp.where(d > 0, nb, 0)
    r = jnp.abs(d)
    is_small = r < max_exact
    rp_safe = jnp.maximum(r, 1).astype(jnp.float32)
    large = max_exact + (
        jnp.log(rp_safe / max_exact)
        / math.log(MAX_DISTANCE / max_exact)
        * (nb - max_exact)
    ).astype(jnp.int32)
    large = jnp.minimum(large, nb - 1)
    bucket = bucket + jnp.where(is_small, r, large)

    bkt = jnp.broadcast_to(bucket, (N_HEADS, C_SRC))
    vtab = jnp.zeros((N_HEADS, C_SRC), jnp.float32)
    for b in range(NUM_BUCKETS):
        vtab = jnp.where(bkt == b, wt_ref[:, b : b + 1], vtab)
    vtab_ref[...] = vtab
    for p in range(N_RHO):
        rho = 8 * p + 7
        for rr in range(8):
            out_ref[:, p, rr, :] = vtab_ref[:, rho - rr : rho - rr + SLAB_W]


def _build_table(W):
    wt = W.T
    return pl.pallas_call(
        _table_body,
        out_shape=jax.ShapeDtypeStruct((N_HEADS, N_RHO, 8, SLAB_W), jnp.float32),
        scratch_shapes=[pltpu.VMEM((N_HEADS, C_SRC), jnp.float32)],
    )(wt)


@functools.lru_cache(maxsize=1)
def _expander():
    mesh = plsc.VectorSubcoreMesh(core_axis_name="c", subcore_axis_name="s")

    @functools.partial(
        pl.kernel,
        mesh=mesh,
        out_type=jax.ShapeDtypeStruct((1, N_HEADS, Q_LEN, K_LEN), jnp.float32),
        scratch_types=[
            pltpu.VMEM((8, SLAB_W), jnp.float32),
            pltpu.VMEM((8, SLAB_W), jnp.float32),
            pltpu.SemaphoreType.DMA,
            pltpu.SemaphoreType.DMA,
            pltpu.SemaphoreType.DMA,
        ],
    )
    def expand(tdb_hbm, out_hbm, slab0, slab1, in_sem, sem0, sem1):
        wid = lax.axis_index("s") * 2 + lax.axis_index("c")
        slabs = (slab0, slab1)
        out_sems = (sem0, sem1)

        def task_hp(n):
            t = wid * 8 + n
            return t // N_RHO, t % N_RHO

        def stage(n):
            h, pidx = task_hp(n)
            return pltpu.async_copy(tdb_hbm.at[h, pidx], slabs[n % 2], in_sem)

        def fire_outs(n):
            h, pidx = task_hp(n)
            copies = []
            for k0 in range(N_K0):
                i0 = pl.multiple_of(8 * (255 - pidx - 16 * k0), 8)
                copies.append(
                    pltpu.async_copy(
                        slabs[n % 2].at[:, pl.ds(128 * k0, K_LEN)],
                        out_hbm.at[0, h, pl.ds(i0, 8), :],
                        out_sems[n % 2],
                    )
                )
            return copies

        ins = {0: stage(0), 1: stage(1)}
        outs = {}
        for n in range(8):
            ins[n].wait()
            outs[n] = fire_outs(n)
            if n + 2 < 8:
                for cp in outs[n]:
                    cp.wait()
                ins[n + 2] = stage(n + 2)
        for n in (6, 7):
            for cp in outs[n]:
                cp.wait()

    return expand


def kernel(q, k, v, W):
    tdb = _build_table(W)
    bias = _expander()(tdb)
    return (q, k, v, bias)
